# Initial kernel scaffold; baseline (speedup 1.0000x reference)
import jax
import jax.numpy as jnp
from jax.experimental import pallas as pl


def kernel(label, log_theta, log_alpha, subgraph_idx, subgraph_idx_base, num_canonical_order):
    raise NotImplementedError("write your pallas kernel here")


# speedup vs baseline: 3.1007x; 3.1007x over previous
import functools

import jax
import jax.numpy as jnp
from jax.experimental import pallas as pl
from jax.experimental.pallas import tpu as pltpu

E = 1048576
K = 20
B = 2048
EC = 2048
NSTEP = E // EC
VC = 2 * K + 1
VP = 48
ACC = 2 * VP


def _bce(logits, y):
    return (jnp.maximum(logits, 0.0) - logits * y
            + jnp.log1p(jnp.exp(-jnp.abs(logits))))


def _seg_kernel(label_ref, theta_ref, alpha_ref, idx_ref, idxn_ref,
                out_ref, acc_ref):
    step = pl.program_id(0)

    theta = theta_ref[...]
    alpha = alpha_ref[...]
    lab = label_ref[...].reshape(EC, 1)
    idx = idx_ref[...]
    m = (idx == idxn_ref[...]).astype(jnp.float32).reshape(EC, 1)

    bce = _bce(theta, lab) * m
    ones = jnp.ones((EC, 1), jnp.float32)
    vals = jnp.concatenate(
        [bce, alpha, ones, jnp.zeros((EC, VP - VC), jnp.float32)], axis=1)
    hi = vals.astype(jnp.bfloat16)
    lo = (vals - hi.astype(jnp.float32)).astype(jnp.bfloat16)
    v2 = jnp.concatenate([hi, lo], axis=1)

    row = jax.lax.broadcasted_iota(jnp.int32, (B, EC), 0)
    oh = (row == idx.reshape(1, EC)).astype(jnp.bfloat16)

    contrib = jnp.dot(oh, v2, preferred_element_type=jnp.float32)

    @pl.when(step == 0)
    def _():
        acc_ref[...] = contrib

    @pl.when(step != 0)
    def _():
        acc_ref[...] += contrib

    @pl.when(step == NSTEP - 1)
    def _():
        acc = acc_ref[...]
        S = acc[:, :VP] + acc[:, VP:]
        nll = S[:, 0:K]
        A = S[:, K:2 * K]
        n = S[:, 2 * K:2 * K + 1]
        ra = A / n
        ra_max = jnp.max(ra, axis=1, keepdims=True)
        ls = ra - ra_max - jnp.log(
            jnp.sum(jnp.exp(ra - ra_max), axis=1, keepdims=True))
        x = -nll + ls
        x_max = jnp.max(x, axis=1, keepdims=True)
        lp = x_max + jnp.log(jnp.sum(jnp.exp(x - x_max), axis=1,
                                     keepdims=True))
        loss_b = -lp / n
        out_ref[...] = jnp.sum(loss_b, axis=0, keepdims=True) / B


@jax.jit
def _run(label, log_theta, log_alpha, subgraph_idx):
    idx = subgraph_idx.astype(jnp.int32)
    idx_next = jnp.concatenate([idx[1:], jnp.full((1,), B, jnp.int32)])
    out = pl.pallas_call(
        _seg_kernel,
        grid=(NSTEP,),
        in_specs=[
            pl.BlockSpec((EC,), lambda i: (i,)),
            pl.BlockSpec((EC, K), lambda i: (i, 0)),
            pl.BlockSpec((EC, K), lambda i: (i, 0)),
            pl.BlockSpec((EC,), lambda i: (i,)),
            pl.BlockSpec((EC,), lambda i: (i,)),
        ],
        out_specs=pl.BlockSpec((1, 1), lambda i: (0, 0)),
        out_shape=jax.ShapeDtypeStruct((1, 1), jnp.float32),
        scratch_shapes=[pltpu.VMEM((B, ACC), jnp.float32)],
    )(label, log_theta, log_alpha, idx, idx_next)
    return out[0, 0]


def kernel(label, log_theta, log_alpha, subgraph_idx, subgraph_idx_base,
           num_canonical_order):
    loss = _run(label, log_theta, log_alpha, subgraph_idx)
    return loss * jnp.asarray(num_canonical_order, jnp.float32)
